# baseline (speedup 1.0000x reference)
import functools
import jax
import jax.numpy as jnp
from jax import lax
from jax.experimental import pallas as pl
from jax.experimental.pallas import tpu as pltpu
from jax.experimental.pallas import tpu_sc as plsc

D = H = W = 128
HW = H * W
NVOX = D * HW
NB = 2
NW = 32
BZ = BY = 8
SZ = 18
SY = 24
SYW = SY * W
MARGIN = 5
NBLK = (D // BZ) * (H // BY)
BPW = NBLK // NW
NROW = BZ * BY
CH = W
G = CH // 16

_mesh = plsc.VectorSubcoreMesh(core_axis_name="c", subcore_axis_name="s")


@functools.partial(
    pl.kernel,
    mesh=_mesh,
    compiler_params=pltpu.CompilerParams(needs_layout_passes=False),
    out_type=jax.ShapeDtypeStruct((NB * 2 * NVOX,), jnp.float32),
    scratch_types=(
        [pltpu.VMEM((SZ * SYW,), jnp.float32)]
        + [pltpu.VMEM((3 * CH,), jnp.float32)] * 2
        + [pltpu.VMEM((2, CH), jnp.int32)]
        + [pltpu.VMEM((CH,), jnp.float32)]
        + [pltpu.VMEM((8, CH), jnp.float32)]
        + [pltpu.VMEM((1, CH), jnp.int32)]
        + [pltpu.VMEM((CH,), jnp.float32)] * 2
        + [pltpu.VMEM((2 * CH,), jnp.float32)] * 2
        + [pltpu.SemaphoreType.DMA] * 6
    ),
)
def _sc_warp(vf, vp, dff, out,
             slab, dfa0, dfa1, loci, xfv, wv, idxf, gf0, gf1,
             ov0, ov1,
             dfsem0, dfsem1, ssem, fsem, osem0, osem1):
    wid = lax.axis_index("s") * 2 + lax.axis_index("c")
    lanes = lax.iota(jnp.int32, 16)
    for g in range(G):
        xfv[pl.ds(g * 16, 16)] = (g * 16 + lanes).astype(jnp.float32)
    dfa = (dfa0, dfa1)
    ov = (ov0, ov1)
    dfsem = (dfsem0, dfsem1)
    osem = (osem0, osem1)

    blk0 = wid * BPW
    off0 = (lax.shift_right_logical(blk0, 4) * BZ * HW
            + jnp.bitwise_and(blk0, 15) * BY * W)
    for s in range(2):
        pltpu.async_copy(ov[s].at[pl.ds(0, CH)],
                         out.at[pl.ds(off0, CH)], osem[s])
        pltpu.async_copy(ov[s].at[pl.ds(CH, CH)],
                         out.at[pl.ds(NVOX + off0, CH)], osem[s])

    def run_block(bv, bd, bvp, zb, yb):
        z0 = jnp.clip(zb - MARGIN, 0, D - SZ)
        y0 = pl.multiple_of(jnp.clip(yb - 8, 0, H - SY), 8)

        hs = []
        for zi in range(SZ):
            hs.append(pltpu.async_copy(
                vp.at[pl.ds(bvp + (z0 + zi) * HW + y0 * W, SYW)],
                slab.at[pl.ds(zi * SYW, SYW)], ssem))

        def rowoff(t):
            z = zb + lax.shift_right_logical(t, 3)
            y = yb + jnp.bitwise_and(t, 7)
            return z, y, z * HW + y * W

        def prefetch(s, t):
            _, _, off = rowoff(t)
            pltpu.async_copy(dff.at[pl.ds(bd + off, CH)],
                             dfa[s].at[pl.ds(0, CH)], dfsem[s])
            pltpu.async_copy(dff.at[pl.ds(bd + NVOX + off, CH)],
                             dfa[s].at[pl.ds(CH, CH)], dfsem[s])
            pltpu.async_copy(dff.at[pl.ds(bd + 2 * NVOX + off, CH)],
                             dfa[s].at[pl.ds(2 * CH, CH)], dfsem[s])

        def df_drain(s):
            pltpu.make_async_copy(dff.at[pl.ds(0, 3 * CH)], dfa[s],
                                  dfsem[s]).wait()

        def out_drain(s):
            pltpu.make_async_copy(out.at[pl.ds(0, 2 * CH)], ov[s],
                                  osem[s]).wait()

        def do_row(s, t):
            z, y, off = rowoff(t)
            df_drain(s)
            out_drain(s)
            zf = z.astype(jnp.float32)
            yf = y.astype(jnp.float32)
            bad = 0
            for g in range(G):
                sl = pl.ds(g * 16, 16)
                locz = jnp.minimum(jnp.maximum(
                    zf + dfa[s][pl.ds(g * 16, 16)], 0.0), 127.0)
                locy = jnp.minimum(jnp.maximum(
                    yf + dfa[s][pl.ds(CH + g * 16, 16)], 0.0), 127.0)
                locx = jnp.minimum(jnp.maximum(
                    xfv[sl] + dfa[s][pl.ds(2 * CH + g * 16, 16)],
                    0.0), 127.0)
                az = jnp.minimum(locz.astype(jnp.int32), 126)
                ay = jnp.minimum(locy.astype(jnp.int32), 126)
                ax = jnp.minimum(locx.astype(jnp.int32), 126)
                tz = locz - az.astype(jnp.float32)
                ty = locy - ay.astype(jnp.float32)
                tx = locx - ax.astype(jnp.float32)
                uz = 1.0 - tz
                uy = 1.0 - ty
                ux = 1.0 - tx
                zi = az - z0
                yi = ay - y0
                ok = ((zi >= 0) & (zi <= SZ - 2)
                      & (yi >= 0) & (yi <= SY - 2))
                bad = bad + (16 - jnp.sum(ok.astype(jnp.int32)))
                fb = jnp.clip(zi * SYW + yi * W + ax, 0,
                              SZ * SYW - SYW - W - 2)
                w00 = uz * uy
                w01 = uz * ty
                w10 = tz * uy
                w11 = tz * ty
                wk = w00 * ux
                pk = plsc.bitcast(plsc.load_gather(slab, [fb]), jnp.int32)
                acc0 = wk * plsc.bitcast(
                    jnp.bitwise_and(pk, jnp.int32(-65536)), jnp.float32)
                acc1 = wk * plsc.bitcast(
                    lax.shift_left(pk, 16), jnp.float32)
                for wgt, o_ in ((w00 * tx, 1),
                                (w01 * ux, W), (w01 * tx, W + 1),
                                (w10 * ux, SYW), (w10 * tx, SYW + 1),
                                (w11 * ux, SYW + W), (w11 * tx, SYW + W + 1)):
                    pk = plsc.bitcast(plsc.load_gather(slab, [fb + o_]),
                                      jnp.int32)
                    acc0 = acc0 + wgt * plsc.bitcast(
                        jnp.bitwise_and(pk, jnp.int32(-65536)), jnp.float32)
                    acc1 = acc1 + wgt * plsc.bitcast(
                        lax.shift_left(pk, 16), jnp.float32)
                ov[s][pl.ds(g * 16, 16)] = acc0
                ov[s][pl.ds(CH + g * 16, 16)] = acc1

            @pl.when(bad != 0)
            def _cold():
                for g in range(G):
                    sl = pl.ds(g * 16, 16)
                    locz = jnp.minimum(jnp.maximum(
                        zf + dfa[s][pl.ds(g * 16, 16)], 0.0), 127.0)
                    locy = jnp.minimum(jnp.maximum(
                        yf + dfa[s][pl.ds(CH + g * 16, 16)], 0.0), 127.0)
                    locx = jnp.minimum(jnp.maximum(
                        xfv[sl] + dfa[s][pl.ds(2 * CH + g * 16, 16)],
                        0.0), 127.0)
                    az = jnp.minimum(locz.astype(jnp.int32), 126)
                    ay = jnp.minimum(locy.astype(jnp.int32), 126)
                    ax = jnp.minimum(locx.astype(jnp.int32), 126)
                    tz = locz - az.astype(jnp.float32)
                    ty = locy - ay.astype(jnp.float32)
                    tx = locx - ax.astype(jnp.float32)
                    uz = 1.0 - tz
                    uy = 1.0 - ty
                    ux = 1.0 - tx
                    loci[1, sl] = (bv + lax.shift_left(az, 14)
                                   + lax.shift_left(ay, 7) + ax)
                    w00 = uz * uy
                    w01 = uz * ty
                    w10 = tz * uy
                    w11 = tz * ty
                    wv[0, sl] = w00 * ux
                    wv[1, sl] = w00 * tx
                    wv[2, sl] = w01 * ux
                    wv[3, sl] = w01 * tx
                    wv[4, sl] = w10 * ux
                    wv[5, sl] = w10 * tx
                    wv[6, sl] = w11 * ux
                    wv[7, sl] = w11 * tx
                for k, o_ in enumerate((0, 1, 128, 129, 16384, 16385,
                                        16512, 16513)):
                    for g in range(G):
                        sl = pl.ds(g * 16, 16)
                        idxf[0, sl] = loci[1, sl] + o_
                    h0 = pltpu.async_copy(vf.at[idxf.at[0]], gf0, fsem)
                    h0.wait()
                    for g in range(G):
                        sl = pl.ds(g * 16, 16)
                        idxf[0, sl] = loci[1, sl] + (o_ + NVOX)
                    h1 = pltpu.async_copy(vf.at[idxf.at[0]], gf1, fsem)
                    h1.wait()
                    for g in range(G):
                        sl = pl.ds(g * 16, 16)
                        sl1 = pl.ds(CH + g * 16, 16)
                        wk = wv[k, sl]
                        if k == 0:
                            ov[s][sl] = wk * gf0[sl]
                            ov[s][sl1] = wk * gf1[sl]
                        else:
                            ov[s][sl] = ov[s][sl] + wk * gf0[sl]
                            ov[s][sl1] = ov[s][sl1] + wk * gf1[sl]

            pltpu.async_copy(ov[s].at[pl.ds(0, CH)],
                             out.at[pl.ds(bv + off, CH)], osem[s])
            pltpu.async_copy(ov[s].at[pl.ds(CH, CH)],
                             out.at[pl.ds(bv + NVOX + off, CH)], osem[s])

        prefetch(0, 0)
        prefetch(1, 1)
        for h in hs:
            h.wait()

        def rows(i, carry):
            t0 = 2 * i
            do_row(0, t0)
            prefetch(0, jnp.minimum(t0 + 2, NROW - 1))
            do_row(1, t0 + 1)
            prefetch(1, jnp.minimum(t0 + 3, NROW - 1))
            return carry

        lax.fori_loop(0, NROW // 2, rows, 0)
        df_drain(0)
        df_drain(1)

    def blocks(i, carry):
        b = lax.shift_right_logical(i, 3)
        blk = wid * BPW + jnp.bitwise_and(i, 7)
        zb = lax.shift_right_logical(blk, 4) * BZ
        yb = jnp.bitwise_and(blk, 15) * BY
        run_block(b * (2 * NVOX), b * (3 * NVOX), b * NVOX, zb, yb)
        return carry

    lax.fori_loop(0, NB * BPW, blocks, 0)

    pltpu.make_async_copy(out.at[pl.ds(0, 2 * CH)], ov0, osem0).wait()
    pltpu.make_async_copy(out.at[pl.ds(0, 2 * CH)], ov1, osem1).wait()


def kernel(vol, df):
    b = vol.reshape(NB, 2, NVOX).astype(jnp.bfloat16)
    pk = jnp.stack([b[:, 1], b[:, 0]], axis=-1)
    vp = lax.bitcast_convert_type(pk, jnp.float32).reshape(NB * NVOX)
    out = _sc_warp(vol.reshape(NB * 2 * NVOX), vp,
                   df.reshape(NB * 3 * NVOX))
    return out.reshape(NB, 2, D, H, W)


# speedup vs baseline: 1.3127x; 1.3127x over previous
import functools
import jax
import jax.numpy as jnp
from jax import lax
from jax.experimental import pallas as pl
from jax.experimental.pallas import tpu as pltpu
from jax.experimental.pallas import tpu_sc as plsc

D = H = W = 128
HW = H * W
NVOX = D * HW
NB = 2
NW = 32
BZ = BY = 8
SZ = 18
SY = 24
SYW = SY * W
MARGIN = 5
NBLK = (D // BZ) * (H // BY)
BPW = NBLK // NW
NROW = BZ * BY
CH = W
G = CH // 16

_mesh = plsc.VectorSubcoreMesh(core_axis_name="c", subcore_axis_name="s")


@functools.partial(
    pl.kernel,
    mesh=_mesh,
    compiler_params=pltpu.CompilerParams(needs_layout_passes=False),
    out_type=jax.ShapeDtypeStruct((NB * 2 * NVOX,), jnp.float32),
    scratch_types=(
        [pltpu.VMEM((SZ * SYW,), jnp.float32)] * 2
        + [pltpu.VMEM((3 * CH,), jnp.float32)] * 2
        + [pltpu.VMEM((2, CH), jnp.int32)]
        + [pltpu.VMEM((CH,), jnp.float32)]
        + [pltpu.VMEM((8, CH), jnp.float32)]
        + [pltpu.VMEM((1, CH), jnp.int32)]
        + [pltpu.VMEM((CH,), jnp.float32)] * 2
        + [pltpu.VMEM((2 * CH,), jnp.float32)] * 2
        + [pltpu.SemaphoreType.DMA] * 6
    ),
)
def _sc_warp(vf, dff, out,
             slab0, slab1, dfa0, dfa1, loci, xfv, wv, idxf, gf0, gf1,
             ov0, ov1,
             dfsem0, dfsem1, ssem, fsem, osem0, osem1):
    wid = lax.axis_index("s") * 2 + lax.axis_index("c")
    lanes = lax.iota(jnp.int32, 16)
    for g in range(G):
        xfv[pl.ds(g * 16, 16)] = (g * 16 + lanes).astype(jnp.float32)
    dfa = (dfa0, dfa1)
    ov = (ov0, ov1)
    dfsem = (dfsem0, dfsem1)
    osem = (osem0, osem1)

    blk0 = wid * BPW
    off0 = (lax.shift_right_logical(blk0, 4) * BZ * HW
            + jnp.bitwise_and(blk0, 15) * BY * W)
    for s in range(2):
        pltpu.async_copy(ov[s].at[pl.ds(0, CH)],
                         out.at[pl.ds(off0, CH)], osem[s])
        pltpu.async_copy(ov[s].at[pl.ds(CH, CH)],
                         out.at[pl.ds(NVOX + off0, CH)], osem[s])

    def run_block(bv, bd, zb, yb):
        z0 = jnp.clip(zb - MARGIN, 0, D - SZ)
        y0 = pl.multiple_of(jnp.clip(yb - 8, 0, H - SY), 8)

        hs = []
        for c in range(2):
            for zi in range(SZ):
                hs.append(pltpu.async_copy(
                    vf.at[pl.ds(bv + c * NVOX + (z0 + zi) * HW + y0 * W,
                                SYW)],
                    (slab0 if c == 0 else slab1).at[pl.ds(zi * SYW, SYW)],
                    ssem))

        def rowoff(t):
            z = zb + lax.shift_right_logical(t, 3)
            y = yb + jnp.bitwise_and(t, 7)
            return z, y, z * HW + y * W

        def prefetch(s, t):
            _, _, off = rowoff(t)
            pltpu.async_copy(dff.at[pl.ds(bd + off, CH)],
                             dfa[s].at[pl.ds(0, CH)], dfsem[s])
            pltpu.async_copy(dff.at[pl.ds(bd + NVOX + off, CH)],
                             dfa[s].at[pl.ds(CH, CH)], dfsem[s])
            pltpu.async_copy(dff.at[pl.ds(bd + 2 * NVOX + off, CH)],
                             dfa[s].at[pl.ds(2 * CH, CH)], dfsem[s])

        def df_drain(s):
            pltpu.make_async_copy(dff.at[pl.ds(0, 3 * CH)], dfa[s],
                                  dfsem[s]).wait()

        def out_drain(s):
            pltpu.make_async_copy(out.at[pl.ds(0, 2 * CH)], ov[s],
                                  osem[s]).wait()

        def do_row(s, t):
            z, y, off = rowoff(t)
            df_drain(s)
            out_drain(s)
            zf = z.astype(jnp.float32)
            yf = y.astype(jnp.float32)
            bad = 0
            for g in range(G):
                sl = pl.ds(g * 16, 16)
                locz = jnp.minimum(jnp.maximum(
                    zf + dfa[s][pl.ds(g * 16, 16)], 0.0), 127.0)
                locy = jnp.minimum(jnp.maximum(
                    yf + dfa[s][pl.ds(CH + g * 16, 16)], 0.0), 127.0)
                locx = jnp.minimum(jnp.maximum(
                    xfv[sl] + dfa[s][pl.ds(2 * CH + g * 16, 16)],
                    0.0), 127.0)
                az = jnp.minimum(locz.astype(jnp.int32), 126)
                ay = jnp.minimum(locy.astype(jnp.int32), 126)
                ax = jnp.minimum(locx.astype(jnp.int32), 126)
                tz = locz - az.astype(jnp.float32)
                ty = locy - ay.astype(jnp.float32)
                tx = locx - ax.astype(jnp.float32)
                uz = 1.0 - tz
                uy = 1.0 - ty
                ux = 1.0 - tx
                zi = az - z0
                yi = ay - y0
                ok = ((zi >= 0) & (zi <= SZ - 2)
                      & (yi >= 0) & (yi <= SY - 2))
                bad = bad + (16 - jnp.sum(ok.astype(jnp.int32)))
                fb = jnp.clip(zi * SYW + yi * W + ax, 0,
                              SZ * SYW - SYW - W - 2)
                w00 = uz * uy
                w01 = uz * ty
                w10 = tz * uy
                w11 = tz * ty
                wk = w00 * ux
                acc0 = wk * plsc.load_gather(slab0, [fb])
                acc1 = wk * plsc.load_gather(slab1, [fb])
                for wgt, o_ in ((w00 * tx, 1),
                                (w01 * ux, W), (w01 * tx, W + 1),
                                (w10 * ux, SYW), (w10 * tx, SYW + 1),
                                (w11 * ux, SYW + W), (w11 * tx, SYW + W + 1)):
                    fo = fb + o_
                    acc0 = acc0 + wgt * plsc.load_gather(slab0, [fo])
                    acc1 = acc1 + wgt * plsc.load_gather(slab1, [fo])
                ov[s][pl.ds(g * 16, 16)] = acc0
                ov[s][pl.ds(CH + g * 16, 16)] = acc1

            @pl.when(bad != 0)
            def _cold():
                for g in range(G):
                    sl = pl.ds(g * 16, 16)
                    locz = jnp.minimum(jnp.maximum(
                        zf + dfa[s][pl.ds(g * 16, 16)], 0.0), 127.0)
                    locy = jnp.minimum(jnp.maximum(
                        yf + dfa[s][pl.ds(CH + g * 16, 16)], 0.0), 127.0)
                    locx = jnp.minimum(jnp.maximum(
                        xfv[sl] + dfa[s][pl.ds(2 * CH + g * 16, 16)],
                        0.0), 127.0)
                    az = jnp.minimum(locz.astype(jnp.int32), 126)
                    ay = jnp.minimum(locy.astype(jnp.int32), 126)
                    ax = jnp.minimum(locx.astype(jnp.int32), 126)
                    tz = locz - az.astype(jnp.float32)
                    ty = locy - ay.astype(jnp.float32)
                    tx = locx - ax.astype(jnp.float32)
                    uz = 1.0 - tz
                    uy = 1.0 - ty
                    ux = 1.0 - tx
                    loci[1, sl] = (bv + lax.shift_left(az, 14)
                                   + lax.shift_left(ay, 7) + ax)
                    w00 = uz * uy
                    w01 = uz * ty
                    w10 = tz * uy
                    w11 = tz * ty
                    wv[0, sl] = w00 * ux
                    wv[1, sl] = w00 * tx
                    wv[2, sl] = w01 * ux
                    wv[3, sl] = w01 * tx
                    wv[4, sl] = w10 * ux
                    wv[5, sl] = w10 * tx
                    wv[6, sl] = w11 * ux
                    wv[7, sl] = w11 * tx
                for k, o_ in enumerate((0, 1, 128, 129, 16384, 16385,
                                        16512, 16513)):
                    for g in range(G):
                        sl = pl.ds(g * 16, 16)
                        idxf[0, sl] = loci[1, sl] + o_
                    h0 = pltpu.async_copy(vf.at[idxf.at[0]], gf0, fsem)
                    h0.wait()
                    for g in range(G):
                        sl = pl.ds(g * 16, 16)
                        idxf[0, sl] = loci[1, sl] + (o_ + NVOX)
                    h1 = pltpu.async_copy(vf.at[idxf.at[0]], gf1, fsem)
                    h1.wait()
                    for g in range(G):
                        sl = pl.ds(g * 16, 16)
                        sl1 = pl.ds(CH + g * 16, 16)
                        wk = wv[k, sl]
                        if k == 0:
                            ov[s][sl] = wk * gf0[sl]
                            ov[s][sl1] = wk * gf1[sl]
                        else:
                            ov[s][sl] = ov[s][sl] + wk * gf0[sl]
                            ov[s][sl1] = ov[s][sl1] + wk * gf1[sl]

            pltpu.async_copy(ov[s].at[pl.ds(0, CH)],
                             out.at[pl.ds(bv + off, CH)], osem[s])
            pltpu.async_copy(ov[s].at[pl.ds(CH, CH)],
                             out.at[pl.ds(bv + NVOX + off, CH)], osem[s])

        prefetch(0, 0)
        prefetch(1, 1)
        for h in hs:
            h.wait()

        def rows(i, carry):
            t0 = 2 * i
            do_row(0, t0)
            prefetch(0, jnp.minimum(t0 + 2, NROW - 1))
            do_row(1, t0 + 1)
            prefetch(1, jnp.minimum(t0 + 3, NROW - 1))
            return carry

        lax.fori_loop(0, NROW // 2, rows, 0)
        df_drain(0)
        df_drain(1)

    def blocks(i, carry):
        b = lax.shift_right_logical(i, 3)
        blk = wid * BPW + jnp.bitwise_and(i, 7)
        zb = lax.shift_right_logical(blk, 4) * BZ
        yb = jnp.bitwise_and(blk, 15) * BY
        run_block(b * (2 * NVOX), b * (3 * NVOX), zb, yb)
        return carry

    lax.fori_loop(0, NB * BPW, blocks, 0)

    pltpu.make_async_copy(out.at[pl.ds(0, 2 * CH)], ov0, osem0).wait()
    pltpu.make_async_copy(out.at[pl.ds(0, 2 * CH)], ov1, osem1).wait()


def kernel(vol, df):
    out = _sc_warp(vol.reshape(NB * 2 * NVOX), df.reshape(NB * 3 * NVOX))
    return out.reshape(NB, 2, D, H, W)
